# Initial kernel scaffold; baseline (speedup 1.0000x reference)
import jax
import jax.numpy as jnp
from jax.experimental import pallas as pl


def kernel(x, W1, b1, W2, b2, W3, b3, Wout, bout, segment_ids):
    raise NotImplementedError("write your pallas kernel here")


# speedup vs baseline: 2.9217x; 2.9217x over previous
import functools

import jax
import jax.numpy as jnp
from jax import lax
from jax.experimental import pallas as pl
from jax.experimental.pallas import tpu as pltpu
from jax.experimental.pallas import tpu_sc as plsc

B = 16
MAXLEN = 4096
N = 32768
D = 256

BLK = 2048
NBLK = N // BLK

ALIGN = 16
WIN = MAXLEN // 2
BUF = WIN + ALIGN
LPAD = N + 2080


def _mlp_body(x_ref, w1_ref, b1_ref, w2_ref, b2_ref, w3_ref, b3_ref,
              wo_ref, bo_ref, seg_ref, logits_ref, counts_ref, starts_ref):
    i = pl.program_id(0)
    h = jnp.maximum(jnp.dot(x_ref[...], w1_ref[...],
                            preferred_element_type=jnp.float32) + b1_ref[...], 0.0)
    h = jnp.maximum(jnp.dot(h, w2_ref[...],
                            preferred_element_type=jnp.float32) + b2_ref[...], 0.0)
    h = jnp.maximum(jnp.dot(h, w3_ref[...],
                            preferred_element_type=jnp.float32) + b3_ref[...], 0.0)
    logits_ref[...] = jnp.dot(h, wo_ref[...],
                              preferred_element_type=jnp.float32) + bo_ref[...]

    seg = seg_ref[0]
    iot = lax.broadcasted_iota(jnp.int32, (B, BLK), 0)
    cnt = jnp.sum((seg == iot).astype(jnp.int32), axis=1, keepdims=True)

    @pl.when(i == 0)
    def _():
        counts_ref[...] = cnt

    @pl.when(i != 0)
    def _():
        counts_ref[...] = counts_ref[...] + cnt

    @pl.when(i == NBLK - 1)
    def _():
        cf = counts_ref[...].astype(jnp.float32)
        row = lax.broadcasted_iota(jnp.int32, (B, B), 0)
        col = lax.broadcasted_iota(jnp.int32, (B, B), 1)
        tril = (col < row).astype(jnp.float32)
        starts_ref[...] = jnp.dot(
            tril, cf, preferred_element_type=jnp.float32).astype(jnp.int32)


_mlp_call = pl.pallas_call(
    _mlp_body,
    grid=(NBLK,),
    in_specs=[
        pl.BlockSpec((BLK, D), lambda i: (i, 0)),
        pl.BlockSpec((D, D), lambda i: (0, 0)),
        pl.BlockSpec((1, D), lambda i: (0, 0)),
        pl.BlockSpec((D, D), lambda i: (0, 0)),
        pl.BlockSpec((1, D), lambda i: (0, 0)),
        pl.BlockSpec((D, D), lambda i: (0, 0)),
        pl.BlockSpec((1, D), lambda i: (0, 0)),
        pl.BlockSpec((D, 1), lambda i: (0, 0)),
        pl.BlockSpec((1, 1), lambda i: (0, 0)),
        pl.BlockSpec((1, 1, BLK), lambda i: (i, 0, 0)),
    ],
    out_specs=[
        pl.BlockSpec((BLK, 1), lambda i: (i, 0)),
        pl.BlockSpec((B, 1), lambda i: (0, 0)),
        pl.BlockSpec((B, 1), lambda i: (0, 0)),
    ],
    out_shape=[
        jax.ShapeDtypeStruct((N, 1), jnp.float32),
        jax.ShapeDtypeStruct((B, 1), jnp.int32),
        jax.ShapeDtypeStruct((B, 1), jnp.int32),
    ],
)


def _sc_pad_body(logits_hbm, starts_hbm, counts_hbm, out_hbm,
                 st_ref, cnt_ref, buf_ref, row_ref):
    wid = lax.axis_index("s") * 2 + lax.axis_index("c")
    b = wid // 2
    off = (wid % 2) * WIN

    pltpu.sync_copy(starts_hbm, st_ref)
    pltpu.sync_copy(counts_hbm, cnt_ref)
    count = jnp.minimum(cnt_ref[pl.ds(b, 1)][0], MAXLEN)
    start = st_ref[pl.ds(b, 1)][0]
    iot = lax.iota(jnp.int32, 16)

    s0 = jnp.minimum(start + off, N)
    aligned = (s0 // ALIGN) * ALIGN
    rem = s0 - aligned
    pltpu.sync_copy(logits_hbm.at[pl.ds(aligned, BUF)], buf_ref)

    def body(j, carry):
        v = buf_ref[pl.ds(rem + j * 16, 16)]
        p = off + j * 16 + iot
        row_ref[pl.ds(j * 16, 16)] = jnp.where(p < count, v, -jnp.inf)
        return carry

    lax.fori_loop(0, WIN // 16, body, 0)
    pltpu.sync_copy(row_ref, out_hbm.at[b, pl.ds(off, WIN)])


@functools.cache
def _sc_pad_call():
    return pl.kernel(
        _sc_pad_body,
        mesh=plsc.VectorSubcoreMesh(core_axis_name="c", subcore_axis_name="s"),
        out_type=jax.ShapeDtypeStruct((B, MAXLEN), jnp.float32),
        scratch_types=[
            pltpu.VMEM((B,), jnp.int32),
            pltpu.VMEM((B,), jnp.int32),
            pltpu.VMEM((BUF,), jnp.float32),
            pltpu.VMEM((WIN,), jnp.float32),
        ],
    )


def kernel(x, W1, b1, W2, b2, W3, b3, Wout, bout, segment_ids):
    seg3d = segment_ids.reshape(NBLK, 1, BLK)
    logits2d, counts2d, starts2d = _mlp_call(
        x, W1, b1.reshape(1, D), W2, b2.reshape(1, D), W3, b3.reshape(1, D),
        Wout, bout.reshape(1, 1), seg3d)
    logits_pad = jnp.concatenate(
        [logits2d[:, 0], jnp.zeros((LPAD - N,), jnp.float32)])
    return _sc_pad_call()(logits_pad, starts2d[:, 0], counts2d[:, 0])


# speedup vs baseline: 3.9745x; 1.3604x over previous
import functools

import jax
import jax.numpy as jnp
from jax import lax
from jax.experimental import pallas as pl
from jax.experimental.pallas import tpu as pltpu
from jax.experimental.pallas import tpu_sc as plsc

B = 16
MAXLEN = 4096
N = 32768
D = 256

BLK = 2048
NBLK = N // BLK

ALIGN = 16
WIN = MAXLEN // 2
BUF = WIN + ALIGN
LPAD = N + 2 * BLK


def _mlp_body(x_ref, w1_ref, b1_ref, w2_ref, b2_ref, w3_ref, b3_ref,
              wo_ref, bo_ref, seg_ref, logits_ref, meta_ref, cnt_acc):
    i = pl.program_id(0)
    h = jnp.maximum(jnp.dot(x_ref[...], w1_ref[...],
                            preferred_element_type=jnp.float32) + b1_ref[...], 0.0)
    h = jnp.maximum(jnp.dot(h, w2_ref[...],
                            preferred_element_type=jnp.float32) + b2_ref[...], 0.0)
    h = jnp.maximum(jnp.dot(h, w3_ref[...],
                            preferred_element_type=jnp.float32) + b3_ref[...], 0.0)
    row = lax.dot_general(wo_ref[...], h, (((0,), (1,)), ((), ())),
                          preferred_element_type=jnp.float32) + bo_ref[...]
    logits_ref[...] = row.reshape(BLK // 128, 128)

    seg = seg_ref[0]
    iot = lax.broadcasted_iota(jnp.int32, (B, BLK), 0)
    cnt = jnp.sum((seg == iot).astype(jnp.int32), axis=1, keepdims=True)

    @pl.when(i == 0)
    def _():
        cnt_acc[...] = cnt

    @pl.when(i != 0)
    def _():
        cnt_acc[...] = cnt_acc[...] + cnt

    @pl.when(i == NBLK - 1)
    def _():
        cf = cnt_acc[...].astype(jnp.float32)
        row_i = lax.broadcasted_iota(jnp.int32, (B, B), 0)
        col_i = lax.broadcasted_iota(jnp.int32, (B, B), 1)
        tril = (col_i < row_i).astype(jnp.float32)
        starts = jnp.dot(tril, cf, preferred_element_type=jnp.float32)
        row0 = jnp.concatenate(
            [starts.reshape(1, B), cf.reshape(1, B),
             jnp.zeros((1, 128 - 2 * B), jnp.float32)], axis=1)
        meta_ref[...] = jnp.concatenate(
            [row0, jnp.zeros((7, 128), jnp.float32)], axis=0)


_mlp_call = pl.pallas_call(
    _mlp_body,
    grid=(NBLK,),
    in_specs=[
        pl.BlockSpec((BLK, D), lambda i: (i, 0)),
        pl.BlockSpec((D, D), lambda i: (0, 0)),
        pl.BlockSpec((1, D), lambda i: (0, 0)),
        pl.BlockSpec((D, D), lambda i: (0, 0)),
        pl.BlockSpec((1, D), lambda i: (0, 0)),
        pl.BlockSpec((D, D), lambda i: (0, 0)),
        pl.BlockSpec((1, D), lambda i: (0, 0)),
        pl.BlockSpec((D, 1), lambda i: (0, 0)),
        pl.BlockSpec((1, 1), lambda i: (0, 0)),
        pl.BlockSpec((1, 1, BLK), lambda i: (i, 0, 0)),
    ],
    out_specs=[
        pl.BlockSpec((BLK // 128, 128), lambda i: (i, 0)),
        pl.BlockSpec((8, 128), lambda i: (0, 0)),
    ],
    out_shape=[
        jax.ShapeDtypeStruct((LPAD // 128, 128), jnp.float32),
        jax.ShapeDtypeStruct((8, 128), jnp.float32),
    ],
    scratch_shapes=[pltpu.VMEM((B, 1), jnp.int32)],
)


def _sc_pad_body(logits_hbm, meta_hbm, out_hbm, meta_ref, buf_ref, row_ref):
    wid = lax.axis_index("s") * 2 + lax.axis_index("c")
    b = wid // 2
    off = (wid % 2) * WIN

    pltpu.sync_copy(meta_hbm.at[pl.ds(0, 32)], meta_ref)
    start = meta_ref[pl.ds(b, 1)][0].astype(jnp.int32)
    count = jnp.minimum(meta_ref[pl.ds(B + b, 1)][0].astype(jnp.int32), MAXLEN)
    iot = lax.iota(jnp.int32, 16)

    s0 = jnp.minimum(start + off, N)
    aligned = (s0 // ALIGN) * ALIGN
    rem = s0 - aligned
    pltpu.sync_copy(logits_hbm.at[pl.ds(aligned, BUF)], buf_ref)

    def body(j, carry):
        v = buf_ref[pl.ds(rem + j * 16, 16)]
        p = off + j * 16 + iot
        row_ref[pl.ds(j * 16, 16)] = jnp.where(p < count, v, -jnp.inf)
        return carry

    lax.fori_loop(0, WIN // 16, body, 0)
    pltpu.sync_copy(row_ref, out_hbm.at[b, pl.ds(off, WIN)])


@functools.cache
def _sc_pad_call():
    return pl.kernel(
        _sc_pad_body,
        mesh=plsc.VectorSubcoreMesh(core_axis_name="c", subcore_axis_name="s"),
        out_type=jax.ShapeDtypeStruct((B, MAXLEN), jnp.float32),
        scratch_types=[
            pltpu.VMEM((32,), jnp.float32),
            pltpu.VMEM((BUF,), jnp.float32),
            pltpu.VMEM((WIN,), jnp.float32),
        ],
    )


def kernel(x, W1, b1, W2, b2, W3, b3, Wout, bout, segment_ids):
    seg3d = segment_ids.reshape(NBLK, 1, BLK)
    logits2d, meta2d = _mlp_call(
        x, W1, b1.reshape(1, D), W2, b2.reshape(1, D), W3, b3.reshape(1, D),
        Wout, bout.reshape(1, 1), seg3d)
    return _sc_pad_call()(logits2d.reshape(LPAD), meta2d.reshape(1024))


# speedup vs baseline: 4.2100x; 1.0593x over previous
import functools

import jax
import jax.numpy as jnp
from jax import lax
from jax.experimental import pallas as pl
from jax.experimental.pallas import tpu as pltpu
from jax.experimental.pallas import tpu_sc as plsc

B = 16
MAXLEN = 4096
N = 32768
D = 256

BLK = 4096
NBLK = N // BLK

ALIGN = 16
WIN = MAXLEN // 2
BUF = WIN + ALIGN
LPAD = N + 2 * BLK


def _mlp_body(x_ref, w1_ref, b1_ref, w2_ref, b2_ref, w3_ref, b3_ref,
              wo_ref, bo_ref, seg_ref, logits_ref, meta_ref, cnt_acc):
    i = pl.program_id(0)
    h = jnp.maximum(jnp.dot(x_ref[...], w1_ref[...],
                            preferred_element_type=jnp.float32) + b1_ref[...], 0.0)
    h = jnp.maximum(jnp.dot(h, w2_ref[...],
                            preferred_element_type=jnp.float32) + b2_ref[...], 0.0)
    h = jnp.maximum(jnp.dot(h, w3_ref[...],
                            preferred_element_type=jnp.float32) + b3_ref[...], 0.0)
    row = lax.dot_general(wo_ref[...], h, (((0,), (1,)), ((), ())),
                          preferred_element_type=jnp.float32) + bo_ref[...]
    logits_ref[...] = row.reshape(BLK // 128, 128)

    seg = seg_ref[0]
    iot = lax.broadcasted_iota(jnp.int32, (B, BLK), 0)
    cnt = jnp.sum((seg == iot).astype(jnp.int32), axis=1, keepdims=True)

    @pl.when(i == 0)
    def _():
        cnt_acc[...] = cnt

    @pl.when(i != 0)
    def _():
        cnt_acc[...] = cnt_acc[...] + cnt

    @pl.when(i == NBLK - 1)
    def _():
        cf = cnt_acc[...].astype(jnp.float32)
        row_i = lax.broadcasted_iota(jnp.int32, (B, B), 0)
        col_i = lax.broadcasted_iota(jnp.int32, (B, B), 1)
        tril = (col_i < row_i).astype(jnp.float32)
        starts = jnp.dot(tril, cf, preferred_element_type=jnp.float32)
        row0 = jnp.concatenate(
            [starts.reshape(1, B), cf.reshape(1, B),
             jnp.zeros((1, 128 - 2 * B), jnp.float32)], axis=1)
        meta_ref[...] = jnp.concatenate(
            [row0, jnp.zeros((7, 128), jnp.float32)], axis=0)


_mlp_call = pl.pallas_call(
    _mlp_body,
    grid=(NBLK,),
    in_specs=[
        pl.BlockSpec((BLK, D), lambda i: (i, 0)),
        pl.BlockSpec((D, D), lambda i: (0, 0)),
        pl.BlockSpec((1, D), lambda i: (0, 0)),
        pl.BlockSpec((D, D), lambda i: (0, 0)),
        pl.BlockSpec((1, D), lambda i: (0, 0)),
        pl.BlockSpec((D, D), lambda i: (0, 0)),
        pl.BlockSpec((1, D), lambda i: (0, 0)),
        pl.BlockSpec((D, 1), lambda i: (0, 0)),
        pl.BlockSpec((1, 1), lambda i: (0, 0)),
        pl.BlockSpec((1, 1, BLK), lambda i: (i, 0, 0)),
    ],
    out_specs=[
        pl.BlockSpec((BLK // 128, 128), lambda i: (i, 0)),
        pl.BlockSpec((8, 128), lambda i: (0, 0)),
    ],
    out_shape=[
        jax.ShapeDtypeStruct((LPAD // 128, 128), jnp.float32),
        jax.ShapeDtypeStruct((8, 128), jnp.float32),
    ],
    scratch_shapes=[pltpu.VMEM((B, 1), jnp.int32)],
)


def _sc_pad_body(logits_hbm, meta_hbm, out_hbm, meta_ref, buf_ref, row_ref):
    wid = lax.axis_index("s") * 2 + lax.axis_index("c")
    b = wid // 2
    off = (wid % 2) * WIN

    pltpu.sync_copy(meta_hbm.at[pl.ds(0, 32)], meta_ref)
    start = meta_ref[pl.ds(b, 1)][0].astype(jnp.int32)
    count = jnp.minimum(meta_ref[pl.ds(B + b, 1)][0].astype(jnp.int32), MAXLEN)
    iot = lax.iota(jnp.int32, 16)

    s0 = jnp.minimum(start + off, N)
    aligned = (s0 // ALIGN) * ALIGN
    rem = s0 - aligned
    pltpu.sync_copy(logits_hbm.at[pl.ds(aligned, BUF)], buf_ref)

    def body(j, carry):
        v = buf_ref[pl.ds(rem + j * 16, 16)]
        p = off + j * 16 + iot
        row_ref[pl.ds(j * 16, 16)] = jnp.where(p < count, v, -jnp.inf)
        return carry

    lax.fori_loop(0, WIN // 16, body, 0)
    pltpu.sync_copy(row_ref, out_hbm.at[b, pl.ds(off, WIN)])


@functools.cache
def _sc_pad_call():
    return pl.kernel(
        _sc_pad_body,
        mesh=plsc.VectorSubcoreMesh(core_axis_name="c", subcore_axis_name="s"),
        out_type=jax.ShapeDtypeStruct((B, MAXLEN), jnp.float32),
        scratch_types=[
            pltpu.VMEM((32,), jnp.float32),
            pltpu.VMEM((BUF,), jnp.float32),
            pltpu.VMEM((WIN,), jnp.float32),
        ],
    )


def kernel(x, W1, b1, W2, b2, W3, b3, Wout, bout, segment_ids):
    seg3d = segment_ids.reshape(NBLK, 1, BLK)
    logits2d, meta2d = _mlp_call(
        x, W1, b1.reshape(1, D), W2, b2.reshape(1, D), W3, b3.reshape(1, D),
        Wout, bout.reshape(1, 1), seg3d)
    return _sc_pad_call()(logits2d.reshape(LPAD), meta2d.reshape(1024))
